# baseline (speedup 1.0000x reference)
import jax
import jax.numpy as jnp
from jax import lax
from jax.experimental import pallas as pl
from jax.experimental.pallas import tpu as pltpu
from jax.experimental.pallas import tpu_sc as plsc

_NC = 2
_NS = 16
_NW = _NC * _NS
_LANES = 16

_BATCH = 16384
_FEAT = 256
_NPAIR = _FEAT // 32
_B_PER_W = _BATCH // _NW
_CHUNK = 64
_NCHUNK = _B_PER_W // _CHUNK
_NBUF = 4
_NACC = 8

_HI_MASK = -65536


def _sc_body(feat_hbm, lab_hbm, cpack_hbm, out_hbm,
             idx_v, feat_v, rows_v, acc_v, fsems, gsems):
    wid = lax.axis_index("s") * _NC + lax.axis_index("c")
    base = wid * _B_PER_W

    pltpu.sync_copy(lab_hbm.at[pl.ds(base, _B_PER_W)], idx_v)

    def issue(ci):
        slot = ci % _NBUF
        off = base + ci * _CHUNK
        g = pltpu.async_copy(
            cpack_hbm.at[idx_v.at[pl.ds(ci * _CHUNK, _CHUNK)]],
            rows_v.at[slot], gsems.at[slot])
        f = pltpu.async_copy(feat_hbm.at[pl.ds(off, _CHUNK), :],
                             feat_v.at[slot], fsems.at[slot])
        return g, f

    pend = {0: issue(0)}
    for _p in range(1, min(3, _NCHUNK)):
        pend[_p] = issue(_p)

    accs = tuple(jnp.zeros((_LANES,), jnp.float32) for _ in range(_NACC))
    for ci in range(_NCHUNK):
        slot = ci % _NBUF
        if ci + 3 < _NCHUNK:
            pend[ci + 3] = issue(ci + 3)
        g, f = pend.pop(ci)
        g.wait()
        f.wait()

        def row_body(i, acc_in, _slot=slot):
            acc_l = list(acc_in)
            for k in range(_NPAIR):
                v = rows_v[_slot, i, pl.ds(k * _LANES, _LANES)]
                c_lo = plsc.bitcast(v << 16, jnp.float32)
                c_hi = plsc.bitcast(v & _HI_MASK, jnp.float32)
                f_lo = feat_v[_slot, i, pl.ds(k * _LANES, _LANES)]
                f_hi = feat_v[_slot, i,
                              pl.ds(_FEAT // 2 + k * _LANES, _LANES)]
                d0 = f_lo - c_lo
                d1 = f_hi - c_hi
                a = 2 * k % _NACC
                acc_l[a] = acc_l[a] + d0 * d0
                acc_l[a + 1] = acc_l[a + 1] + d1 * d1
            return tuple(acc_l)

        accs = lax.fori_loop(0, _CHUNK, row_body, accs)

    total = accs[0]
    for a in accs[1:]:
        total = total + a
    acc_v[...] = total
    pltpu.sync_copy(acc_v, out_hbm.at[wid])


@jax.jit
def kernel(features, labels, centers):
    labels = labels.astype(jnp.int32)
    u = lax.bitcast_convert_type(centers, jnp.int32)
    r = u + 0x7FFF + ((u >> 16) & 1)
    cpack = (((r[:, :_FEAT // 2] >> 16) & 0xFFFF)
             | (r[:, _FEAT // 2:] & _HI_MASK))
    mesh = plsc.VectorSubcoreMesh(core_axis_name="c", subcore_axis_name="s")
    partial = pl.kernel(
        _sc_body,
        out_type=jax.ShapeDtypeStruct((_NW, _LANES), jnp.float32),
        mesh=mesh,
        compiler_params=pltpu.CompilerParams(needs_layout_passes=False),
        scratch_types=[
            pltpu.VMEM((_B_PER_W,), jnp.int32),
            pltpu.VMEM((_NBUF, _CHUNK, _FEAT), jnp.float32),
            pltpu.VMEM((_NBUF, _CHUNK, _FEAT // 2), jnp.int32),
            pltpu.VMEM((_LANES,), jnp.float32),
            pltpu.SemaphoreType.DMA((_NBUF,)),
            pltpu.SemaphoreType.DMA((_NBUF,)),
        ],
    )(features, labels, cpack)
    return jnp.sum(partial) / 2.0 / features.shape[0]
